# baseline (speedup 1.0000x reference)
import functools

import jax
import jax.numpy as jnp
from jax import lax
from jax.experimental import pallas as pl
from jax.experimental.pallas import tpu as pltpu
from jax.experimental.pallas import tpu_sc as plsc

BATCH = 4096
HIST = 200
DIM = 64
NUM_CORES = 2
NUM_SUBCORES = 16
NW = NUM_CORES * NUM_SUBCORES
ROWS_W = BATCH // NW
G = 4
N_GROUPS = ROWS_W // G
SPLIT = 128
REST = HIST - SPLIT


def _emb_body(idx_hbm, table_hbm, out_hbm, idx_v, rows_v, gsem, ssem):
    cid = lax.axis_index("c")
    sid = lax.axis_index("s")
    wid = sid * NUM_CORES + cid
    base_b = wid * ROWS_W

    def stage_idx(g, half):
        pltpu.sync_copy(idx_hbm.at[wid, pl.ds(g * G, G)], idx_v.at[half])

    def fire_gathers(half):
        for r in range(G):
            pltpu.async_copy(
                table_hbm.at[idx_v.at[half, r, pl.ds(0, SPLIT)]],
                rows_v.at[half, r, pl.ds(0, SPLIT)], gsem)
            pltpu.async_copy(
                table_hbm.at[idx_v.at[half, r, pl.ds(SPLIT, REST)]],
                rows_v.at[half, r, pl.ds(SPLIT, REST)], gsem)

    def drain_gathers(half):
        for r in range(G):
            pltpu.make_async_copy(
                table_hbm.at[pl.ds(0, SPLIT)],
                rows_v.at[half, r, pl.ds(0, SPLIT)], gsem).wait()
            pltpu.make_async_copy(
                table_hbm.at[pl.ds(0, REST)],
                rows_v.at[half, r, pl.ds(SPLIT, REST)], gsem).wait()

    def fire_scatters(g, half):
        for r in range(G):
            pltpu.async_copy(
                rows_v.at[half, r], out_hbm.at[base_b + g * G + r], ssem)

    def drain_scatters(half):
        for r in range(G):
            pltpu.make_async_copy(
                rows_v.at[half, r], out_hbm.at[base_b], ssem).wait()

    stage_idx(0, 0)
    fire_gathers(0)

    def body(g, carry):
        half = lax.rem(g, 2)
        other = 1 - half
        drain_gathers(half)

        @pl.when(g >= 1)
        def _():
            drain_scatters(other)

        @pl.when(g < N_GROUPS - 1)
        def _():
            stage_idx(g + 1, other)
            fire_gathers(other)

        fire_scatters(g, half)
        return carry

    lax.fori_loop(0, N_GROUPS, body, 0)
    drain_scatters((N_GROUPS - 1) % 2)


@functools.partial(
    pl.kernel,
    mesh=plsc.VectorSubcoreMesh(core_axis_name="c", subcore_axis_name="s"),
    compiler_params=pltpu.CompilerParams(use_tc_tiling_on_sc=False),
    out_type=jax.ShapeDtypeStruct((BATCH, HIST, DIM), jnp.float32),
    scratch_types=[
        pltpu.VMEM((2, G, HIST), jnp.int32),
        pltpu.VMEM((2, G, HIST, DIM), jnp.float32),
        pltpu.SemaphoreType.DMA,
        pltpu.SemaphoreType.DMA,
    ],
)
def _gather(idx_hbm, table_hbm, out_hbm, idx_v, rows_v, gsem, ssem):
    _emb_body(idx_hbm, table_hbm, out_hbm, idx_v, rows_v, gsem, ssem)


def kernel(skill_ids, embeddings):
    idx = skill_ids.reshape(NW, ROWS_W, HIST).astype(jnp.int32)
    return _gather(idx, embeddings)
